# Initial kernel scaffold; baseline (speedup 1.0000x reference)
import jax
import jax.numpy as jnp
from jax.experimental import pallas as pl


def kernel(x):
    raise NotImplementedError("write your pallas kernel here")


# speedup vs baseline: 26.4168x; 26.4168x over previous
import jax
import jax.numpy as jnp
from jax import lax
from jax.experimental import pallas as pl
from jax.experimental.pallas import tpu as pltpu

_LEVELS = 16
_OFFSETS = ((0, 5), (-5, 5), (-5, 0), (-5, -5))
_W1SUM = 728.0
_NPAIR = 14.0


def _glcm_kernel(x_ref, o_ref):
    img = x_ref[0, 0]
    H, W = img.shape

    mn = jnp.min(img)
    mx = jnp.max(img)
    q = (_LEVELS - 1) * (img - mn) / (mx - mn) + 1.0
    r = jnp.round(q)
    mc = (q == r) & (r >= 1.0) & (r <= float(_LEVELS))
    mcf = mc.astype(jnp.float32)
    a = (r - 1.0) * mcf

    s = 2.0 * a - 18.0
    w2 = (a >= 3.0).astype(jnp.float32)
    w1 = s * s * w2
    pc1 = mcf * w1
    pc2 = mcf * w2
    t1 = jnp.sum(pc1)
    t2 = jnp.sum(pc2)

    rows = lax.broadcasted_iota(jnp.int32, (H, W), 0)
    cols = lax.broadcasted_iota(jnp.int32, (H, W), 1)

    def _shift(arr, dy, dx):
        if dy:
            arr = jnp.roll(arr, -dy, axis=0)
        if dx:
            arr = jnp.roll(arr, -dx, axis=1)
        return arr

    feats = []
    for dy, dx in _OFFSETS:
        an = _shift(a, dy, dx)
        mnf = _shift(mcf, dy, dx)
        valid = jnp.ones((H, W), jnp.float32)
        if dy:
            valid = valid * ((rows + dy >= 0) & (rows + dy < H)).astype(jnp.float32)
        if dx:
            valid = valid * ((cols + dx >= 0) & (cols + dx < W)).astype(jnp.float32)
        pair = valid * mcf * mnf
        d = a - an
        s_num = jnp.sum(pair * d * d)
        s_cnt = jnp.sum(pair)
        sv1 = jnp.sum(valid * pc1)
        sv2 = jnp.sum(valid * pc2)
        pt = float(H * W - (H - abs(dy)) * (W - abs(dx)))
        num = 2.0 * s_num + _W1SUM * pt - 2.0 * (t1 - sv1)
        den = 2.0 * s_cnt + _NPAIR * pt - 2.0 * (t2 - sv2)
        feats.append(num / den)

    o_ref[:, :, :] = jnp.stack(feats).reshape(1, 1, len(_OFFSETS))


def kernel(x):
    B, C, H, W = x.shape
    out = pl.pallas_call(
        _glcm_kernel,
        grid=(B,),
        in_specs=[pl.BlockSpec((1, C, H, W), lambda i: (i, 0, 0, 0))],
        out_specs=pl.BlockSpec((1, 1, len(_OFFSETS)), lambda i: (i, 0, 0)),
        out_shape=jax.ShapeDtypeStruct((B, 1, len(_OFFSETS)), jnp.float32),
        compiler_params=pltpu.CompilerParams(
            dimension_semantics=("parallel",),
        ),
    )(x)
    return out.reshape(B, 1, 1, len(_OFFSETS))
